# Initial kernel scaffold; baseline (speedup 1.0000x reference)
import jax
import jax.numpy as jnp
from jax.experimental import pallas as pl


def kernel(x, edge_index, edge_attr, batch, lin_in_W, lin_in_b, msg_W, msg_b, upd_W, upd_b, pred_W, pred_b):
    raise NotImplementedError("write your pallas kernel here")


# speedup vs baseline: 3.0203x; 3.0203x over previous
import functools

import jax
import jax.numpy as jnp
from jax import lax
from jax.experimental import pallas as pl
from jax.experimental.pallas import tpu as pltpu
from jax.experimental.pallas import tpu_sc as plsc

N = 100000
E = 1600000
D = 32
DE = 4
L = 4
G = 128

NC = 2
NS = 16
K = 128
CPT = 782
EPT = CPT * K
E_PAD = NS * EPT

H = 50048
N_PAD = NC * H
SPM_ROWS = H + 8
ZROWS = H // NS

BN = 2000
NBLK = N // BN


def _precise_dot(a, b):
    return jnp.dot(a, b, precision=lax.Precision.HIGHEST,
                   preferred_element_type=jnp.float32)


def _edge_body(a_hbm, b_hbm, src_hbm, dst_hbm, ea_hbm, we_hbm, mb_hbm,
               z_hbm, agg_hbm, src_v, dst_v, dloc_v, ea_v, a_v, b_v, m_v,
               we_v, mb_v, agg_sh, sem_a, sem_b):
    c = lax.axis_index("c")
    s = lax.axis_index("s")
    hbase = c * H

    pltpu.sync_copy(we_hbm, we_v)
    pltpu.sync_copy(mb_hbm, mb_v)
    pltpu.sync_copy(z_hbm, agg_sh.at[pl.ds(s * ZROWS, ZROWS), :])
    plsc.subcore_barrier()

    w0 = [we_v[k, pl.ds(0, 16)] for k in range(DE)]
    w1 = [we_v[k, pl.ds(16, 16)] for k in range(DE)]
    mb0 = mb_v[pl.ds(0, 16)]
    mb1 = mb_v[pl.ds(16, 16)]

    def chunk(i, _):
        base = pl.multiple_of(s * EPT + i * K, K)
        pltpu.sync_copy(src_hbm.at[pl.ds(base, K)], src_v)
        pltpu.sync_copy(dst_hbm.at[pl.ds(base, K)], dst_v)
        pltpu.sync_copy(ea_hbm.at[pl.ds(base * DE, K * DE)], ea_v)
        ga = pltpu.async_copy(a_hbm.at[src_v], a_v, sem_a)
        gb = pltpu.async_copy(b_hbm.at[dst_v], b_v, sem_b)
        ga.wait()
        gb.wait()

        def dl_body(j, _):
            d = dst_v[pl.ds(j * 16, 16)]
            dl = d - hbase
            ok = (dl >= 0) & (dl < H)
            dloc_v[pl.ds(j * 16, 16)] = jnp.where(ok, dl, H)
            return 0

        lax.fori_loop(0, K // 16, dl_body, 0, unroll=True)

        def g_body(g, _):
            av = ea_v[pl.ds(g * 16, 16)]
            for j in range(4):
                e = g * 4 + j
                e0 = av[j * DE]
                e1 = av[j * DE + 1]
                e2 = av[j * DE + 2]
                e3 = av[j * DE + 3]
                m0 = (a_v[e, pl.ds(0, 16)] + b_v[e, pl.ds(0, 16)] + mb0
                      + e0 * w0[0] + e1 * w0[1] + e2 * w0[2] + e3 * w0[3])
                m_v[e, pl.ds(0, 16)] = jnp.maximum(m0, 0.0)
                m1 = (a_v[e, pl.ds(16, 16)] + b_v[e, pl.ds(16, 16)] + mb1
                      + e0 * w1[0] + e1 * w1[1] + e2 * w1[2] + e3 * w1[3])
                m_v[e, pl.ds(16, 16)] = jnp.maximum(m1, 0.0)
            return 0

        lax.fori_loop(0, K // 4, g_body, 0)
        pltpu.sync_copy(m_v, agg_sh.at[dloc_v], add=True)
        return 0

    lax.fori_loop(0, CPT, chunk, 0)
    plsc.subcore_barrier()
    pltpu.sync_copy(agg_sh.at[pl.ds(s * ZROWS, ZROWS), :],
                    agg_hbm.at[pl.ds(hbase + s * ZROWS, ZROWS), :])


_edge_kernel = functools.partial(
    pl.kernel,
    _edge_body,
    out_type=jax.ShapeDtypeStruct((N_PAD, D), jnp.float32),
    mesh=plsc.VectorSubcoreMesh(core_axis_name="c", subcore_axis_name="s"),
    scratch_types=[
        pltpu.VMEM((K,), jnp.int32),
        pltpu.VMEM((K,), jnp.int32),
        pltpu.VMEM((K,), jnp.int32),
        pltpu.VMEM((K * DE,), jnp.float32),
        pltpu.VMEM((K, D), jnp.float32),
        pltpu.VMEM((K, D), jnp.float32),
        pltpu.VMEM((K, D), jnp.float32),
        pltpu.VMEM((DE, D), jnp.float32),
        pltpu.VMEM((D,), jnp.float32),
        pltpu.VMEM_SHARED((SPM_ROWS, D), jnp.float32),
        pltpu.SemaphoreType.DMA,
        pltpu.SemaphoreType.DMA,
    ],
    compiler_params=pltpu.CompilerParams(use_tc_tiling_on_sc=False),
)()


def _pre_body(x_ref, lw_ref, lb_ref, ws_ref, wd_ref, h_ref, a_ref, b_ref):
    h = _precise_dot(x_ref[...], lw_ref[...]) + lb_ref[...]
    h_ref[...] = h
    a_ref[...] = _precise_dot(h, ws_ref[...])
    b_ref[...] = _precise_dot(h, wd_ref[...])


def _upd_body(with_next, h_ref, agg_ref, w1_ref, w2_ref, ub_ref, ws_ref,
              wd_ref, h_ref_o, a_ref=None, b_ref=None):
    h = h_ref[...]
    u = jnp.maximum(_precise_dot(h, w1_ref[...])
                    + _precise_dot(agg_ref[...], w2_ref[...])
                    + ub_ref[...], 0.0)
    hn = h + u
    h_ref_o[...] = hn
    if with_next:
        a_ref[...] = _precise_dot(hn, ws_ref[...])
        b_ref[...] = _precise_dot(hn, wd_ref[...])


def _pool_body(h_ref, bat_ref, pw_ref, pb_ref, out_ref, acc_ref, cnt_ref):
    i = pl.program_id(0)

    @pl.when(i == 0)
    def _():
        acc_ref[...] = jnp.zeros_like(acc_ref)
        cnt_ref[...] = jnp.zeros_like(cnt_ref)

    bb = bat_ref[0, 0, :]
    gi = lax.broadcasted_iota(jnp.int32, (BN, G), 1)
    oh = (bb[:, None] == gi).astype(jnp.float32)
    acc_ref[...] += lax.dot_general(
        oh, h_ref[...], (((0,), (0,)), ((), ())),
        precision=lax.Precision.HIGHEST, preferred_element_type=jnp.float32)
    cnt_ref[...] += jnp.sum(oh, axis=0).reshape(1, G)

    @pl.when(i == NBLK - 1)
    def _():
        cnts = jnp.maximum(cnt_ref[0, :], 1.0)
        hg = acc_ref[...] / cnts[:, None]
        out_ref[0, :] = jnp.sum(hg * pw_ref[...], axis=1) + pb_ref[0, 0]


_row_spec = pl.BlockSpec((BN, D), lambda i: (i, 0))
_w_spec = pl.BlockSpec((D, D), lambda i: (0, 0))
_b_spec = pl.BlockSpec((1, D), lambda i: (0, 0))
_f32 = jnp.float32


def _tc_pre(x, lw, lb, ws, wd):
    return pl.pallas_call(
        _pre_body,
        grid=(NBLK,),
        in_specs=[_row_spec, _w_spec, _b_spec, _w_spec, _w_spec],
        out_specs=[_row_spec, _row_spec, _row_spec],
        out_shape=[jax.ShapeDtypeStruct((N, D), _f32)] * 3,
    )(x, lw, lb, ws, wd)


def _tc_update(h, agg, w1, w2, ub, ws, wd, with_next):
    n_out = 3 if with_next else 1
    out = pl.pallas_call(
        functools.partial(_upd_body, with_next),
        grid=(NBLK,),
        in_specs=[_row_spec, _row_spec, _w_spec, _w_spec, _b_spec,
                  _w_spec, _w_spec],
        out_specs=[_row_spec] * n_out,
        out_shape=[jax.ShapeDtypeStruct((N, D), _f32)] * n_out,
    )(h, agg, w1, w2, ub, ws, wd)
    return out if with_next else (out[0], None, None)


def _tc_pool(h, batch3, pw, pb):
    return pl.pallas_call(
        _pool_body,
        grid=(NBLK,),
        in_specs=[_row_spec,
                  pl.BlockSpec((1, 1, BN), lambda i: (i, 0, 0)),
                  _b_spec,
                  pl.BlockSpec((1, 1), lambda i: (0, 0))],
        out_specs=pl.BlockSpec((1, G), lambda i: (0, 0)),
        out_shape=jax.ShapeDtypeStruct((1, G), _f32),
        scratch_shapes=[pltpu.VMEM((G, D), _f32), pltpu.VMEM((1, G), _f32)],
    )(h, batch3, pw, pb)


def kernel(x, edge_index, edge_attr, batch, lin_in_W, lin_in_b, msg_W,
           msg_b, upd_W, upd_b, pred_W, pred_b):
    src = edge_index[0]
    dst = edge_index[1]
    pad = E_PAD - E
    src_p = jnp.concatenate([src, jnp.zeros((pad,), jnp.int32)])
    dst_p = jnp.concatenate([dst, jnp.full((pad,), N, jnp.int32)])
    ea_p = jnp.concatenate(
        [edge_attr, jnp.zeros((pad, DE), jnp.float32)]).reshape(-1)
    zeros_tile = jnp.zeros((ZROWS, D), jnp.float32)
    batch3 = batch.reshape(NBLK, 1, BN)

    h, a, b = _tc_pre(x, lin_in_W, lin_in_b.reshape(1, D),
                      msg_W[0][:D], msg_W[0][D:2 * D])
    for l in range(L):
        agg = _edge_kernel(a, b, src_p, dst_p, ea_p, msg_W[l][2 * D:],
                           msg_b[l], zeros_tile)
        with_next = l + 1 < L
        nl = l + 1 if with_next else l
        h, a, b = _tc_update(h, agg, upd_W[l][:D], upd_W[l][D:],
                             upd_b[l].reshape(1, D), msg_W[nl][:D],
                             msg_W[nl][D:2 * D], with_next)
    out = _tc_pool(h, batch3, pred_W.reshape(1, D),
                   pred_b.reshape(1, 1))
    return out.reshape(-1)
